# Initial kernel scaffold; baseline (speedup 1.0000x reference)
import jax
import jax.numpy as jnp
from jax.experimental import pallas as pl


def kernel(input_ids, type_ids, word_table, pos_emb, type_table, ln_scale, ln_bias, W, b):
    raise NotImplementedError("write your pallas kernel here")


# speedup vs baseline: 1.2786x; 1.2786x over previous
import functools

import jax
import jax.numpy as jnp
from jax import lax
from jax.experimental import pallas as pl
from jax.experimental.pallas import tpu as pltpu
from jax.experimental.pallas import tpu_sc as plsc

_B, _S, _V, _D, _T, _M = 4, 2048, 100000, 128, 2, 768
_LN_EPS = 1e-12
_IDX_CHUNK = 128


def _sc_gather(table, idx):
  n = idx.shape[0]
  d = table.shape[1]
  info = plsc.get_sparse_core_info()
  nw = info.num_cores * info.num_subcores
  per_w = n // nw
  n_chunks = per_w // _IDX_CHUNK
  mesh = plsc.VectorSubcoreMesh(core_axis_name="c", subcore_axis_name="s")

  @functools.partial(
      pl.kernel,
      mesh=mesh,
      out_type=jax.ShapeDtypeStruct((n, d), jnp.float32),
      scratch_types=[
          pltpu.VMEM((n_chunks, _IDX_CHUNK), jnp.int32),
          pltpu.VMEM((per_w, d), jnp.float32),
          pltpu.SemaphoreType.DMA,
      ],
  )
  def k(table_hbm, idx_hbm, out_hbm, idx_v, rows_v, sem):
    wid = lax.axis_index("s") * info.num_cores + lax.axis_index("c")
    base = wid * per_w
    for j in range(n_chunks):
      pltpu.sync_copy(idx_hbm.at[pl.ds(base + j * _IDX_CHUNK, _IDX_CHUNK)],
                      idx_v.at[j])
    copies = [
        pltpu.async_copy(table_hbm.at[idx_v.at[j]],
                         rows_v.at[pl.ds(j * _IDX_CHUNK, _IDX_CHUNK)], sem)
        for j in range(n_chunks)
    ]
    for c in copies:
      c.wait()
    pltpu.sync_copy(rows_v, out_hbm.at[pl.ds(base, per_w)])

  return k(table, idx)


def _dense_body(w_ref, pos_ref, tf_ref, tt_ref, ls_ref, lb_ref, W_ref, b_ref,
                o_ref):
  tf = tf_ref[...]
  tt0 = tt_ref[0:1, :]
  tt1 = tt_ref[1:2, :]
  x = w_ref[...] + pos_ref[...] + (tt0 + tf * (tt1 - tt0))
  mean = jnp.mean(x, axis=-1, keepdims=True)
  xc = x - mean
  var = jnp.mean(xc * xc, axis=-1, keepdims=True)
  normed = xc * lax.rsqrt(var + _LN_EPS)
  normed = normed * ls_ref[...] + lb_ref[...]
  o_ref[...] = (
      jnp.dot(normed, W_ref[...], preferred_element_type=jnp.float32)
      + b_ref[...])


def _tc_dense(wrows, pos2d, type_f, type_table, ln_scale, ln_bias, W, b, blk):
  n = wrows.shape[0]
  grid = (n // blk,)
  s_blocks = _S // blk
  return pl.pallas_call(
      _dense_body,
      grid=grid,
      in_specs=[
          pl.BlockSpec((blk, _D), lambda i: (i, 0)),
          pl.BlockSpec((blk, _D), lambda i: (i % s_blocks, 0)),
          pl.BlockSpec((blk, 1), lambda i: (i, 0)),
          pl.BlockSpec((_T, _D), lambda i: (0, 0)),
          pl.BlockSpec((1, _D), lambda i: (0, 0)),
          pl.BlockSpec((1, _D), lambda i: (0, 0)),
          pl.BlockSpec((_D, _M), lambda i: (0, 0)),
          pl.BlockSpec((1, _M), lambda i: (0, 0)),
      ],
      out_specs=pl.BlockSpec((blk, _M), lambda i: (i, 0)),
      out_shape=jax.ShapeDtypeStruct((n, _M), jnp.float32),
      compiler_params=pltpu.CompilerParams(
          dimension_semantics=("arbitrary",)),
  )(wrows, pos2d, type_f, type_table, ln_scale, ln_bias, W, b)


def kernel(input_ids, type_ids, word_table, pos_emb, type_table, ln_scale,
           ln_bias, W, b):
  bs = _B * _S
  ids_flat = input_ids.reshape(bs)
  wrows = _sc_gather(word_table, ids_flat)
  pos2d = pos_emb.reshape(_S, _D)
  type_f = type_ids.reshape(bs, 1).astype(jnp.float32)
  out = _tc_dense(wrows, pos2d, type_f, type_table, ln_scale.reshape(1, _D),
                  ln_bias.reshape(1, _D), W, b.reshape(1, _M), blk=512)
  return out.reshape(_B, _S, _M)


# speedup vs baseline: 1.3965x; 1.0922x over previous
import functools

import jax
import jax.numpy as jnp
from jax import lax
from jax.experimental import pallas as pl
from jax.experimental.pallas import tpu as pltpu
from jax.experimental.pallas import tpu_sc as plsc

_B, _S, _V, _D, _T, _M = 4, 2048, 100000, 128, 2, 768
_LN_EPS = 1e-12
_IDX_CHUNK = 128


def _sc_gather(table, idx):
  n = idx.shape[0]
  d = table.shape[1]
  info = plsc.get_sparse_core_info()
  nw = info.num_cores * info.num_subcores
  per_w = n // nw
  n_chunks = per_w // _IDX_CHUNK
  mesh = plsc.VectorSubcoreMesh(core_axis_name="c", subcore_axis_name="s")

  @functools.partial(
      pl.kernel,
      mesh=mesh,
      out_type=jax.ShapeDtypeStruct((n, d), jnp.float32),
      scratch_types=[
          pltpu.VMEM((n_chunks, _IDX_CHUNK), jnp.int32),
          pltpu.VMEM((per_w, d), jnp.float32),
          pltpu.SemaphoreType.DMA,
      ],
  )
  def k(table_hbm, idx_hbm, out_hbm, idx_v, rows_v, sem):
    wid = lax.axis_index("s") * info.num_cores + lax.axis_index("c")
    base = wid * per_w
    for j in range(n_chunks):
      pltpu.sync_copy(idx_hbm.at[pl.ds(base + j * _IDX_CHUNK, _IDX_CHUNK)],
                      idx_v.at[j])
    copies = [
        pltpu.async_copy(table_hbm.at[idx_v.at[j]],
                         rows_v.at[pl.ds(j * _IDX_CHUNK, _IDX_CHUNK)], sem)
        for j in range(n_chunks)
    ]
    for c in copies:
      c.wait()
    pltpu.sync_copy(rows_v, out_hbm.at[pl.ds(base, per_w)])

  return k(table, idx)


def _dense_body(w_ref, pos_ref, tf_ref, tt_ref, ls_ref, lb_ref, W_ref, b_ref,
                o_ref):
  tf = tf_ref[...]
  tt0 = tt_ref[0:1, :]
  tt1 = tt_ref[1:2, :]
  x = w_ref[...] + pos_ref[...] + (tt0 + tf * (tt1 - tt0))
  mean = jnp.mean(x, axis=-1, keepdims=True)
  xc = x - mean
  var = jnp.mean(xc * xc, axis=-1, keepdims=True)
  normed = xc * lax.rsqrt(var + _LN_EPS)
  normed = normed * ls_ref[...] + lb_ref[...]
  o_ref[...] = (
      jnp.dot(normed, W_ref[...], preferred_element_type=jnp.float32)
      + b_ref[...])


def _tc_dense(wrows, pos2d, type_f, type_table, ln_scale, ln_bias, W, b, blk):
  n = wrows.shape[0]
  grid = (n // blk,)
  s_blocks = _S // blk
  return pl.pallas_call(
      _dense_body,
      grid=grid,
      in_specs=[
          pl.BlockSpec((blk, _D), lambda i: (i, 0)),
          pl.BlockSpec((blk, _D), lambda i: (i % s_blocks, 0)),
          pl.BlockSpec((blk, 1), lambda i: (i, 0)),
          pl.BlockSpec((_T, _D), lambda i: (0, 0)),
          pl.BlockSpec((1, _D), lambda i: (0, 0)),
          pl.BlockSpec((1, _D), lambda i: (0, 0)),
          pl.BlockSpec((_D, _M), lambda i: (0, 0)),
          pl.BlockSpec((1, _M), lambda i: (0, 0)),
      ],
      out_specs=pl.BlockSpec((blk, _M), lambda i: (i, 0)),
      out_shape=jax.ShapeDtypeStruct((n, _M), jnp.float32),
      compiler_params=pltpu.CompilerParams(
          dimension_semantics=("arbitrary",)),
  )(wrows, pos2d, type_f, type_table, ln_scale, ln_bias, W, b)


def kernel(input_ids, type_ids, word_table, pos_emb, type_table, ln_scale,
           ln_bias, W, b):
  bs = _B * _S
  ids_flat = input_ids.reshape(bs)
  wrows = _sc_gather(word_table, ids_flat)
  pos2d = pos_emb.reshape(_S, _D)
  type_f = type_ids.reshape(bs, 1).astype(jnp.float32)
  out = _tc_dense(wrows, pos2d, type_f, type_table, ln_scale.reshape(1, _D),
                  ln_bias.reshape(1, _D), W, b.reshape(1, _M), blk=1024)
  return out.reshape(_B, _S, _M)


# speedup vs baseline: 1.4916x; 1.0681x over previous
import functools

import jax
import jax.numpy as jnp
from jax import lax
from jax.experimental import pallas as pl
from jax.experimental.pallas import tpu as pltpu
from jax.experimental.pallas import tpu_sc as plsc

_B, _S, _V, _D, _T, _M = 4, 2048, 100000, 128, 2, 768
_LN_EPS = 1e-12
_IDX_CHUNK = 128


def _sc_gather(table, idx):
  n = idx.shape[0]
  d = table.shape[1]
  info = plsc.get_sparse_core_info()
  nw = info.num_cores * info.num_subcores
  per_w = n // nw
  n_chunks = per_w // _IDX_CHUNK
  mesh = plsc.VectorSubcoreMesh(core_axis_name="c", subcore_axis_name="s")

  @functools.partial(
      pl.kernel,
      mesh=mesh,
      out_type=jax.ShapeDtypeStruct((n, d), jnp.float32),
      scratch_types=[
          pltpu.VMEM((n_chunks, _IDX_CHUNK), jnp.int32),
          pltpu.VMEM((per_w, d), jnp.float32),
          pltpu.SemaphoreType.DMA,
      ],
  )
  def k(table_hbm, idx_hbm, out_hbm, idx_v, rows_v, sem):
    wid = lax.axis_index("s") * info.num_cores + lax.axis_index("c")
    base = wid * per_w
    for j in range(n_chunks):
      pltpu.sync_copy(idx_hbm.at[pl.ds(base + j * _IDX_CHUNK, _IDX_CHUNK)],
                      idx_v.at[j])
    copies = [
        pltpu.async_copy(table_hbm.at[idx_v.at[j]],
                         rows_v.at[pl.ds(j * _IDX_CHUNK, _IDX_CHUNK)], sem)
        for j in range(n_chunks)
    ]
    for c in copies:
      c.wait()
    pltpu.sync_copy(rows_v, out_hbm.at[pl.ds(base, per_w)])

  return k(table, idx)


def _dense_body(w_ref, pos_ref, tf_ref, tt_ref, ls_ref, lb_ref, W_ref, b_ref,
                o_ref):
  tf = tf_ref[...]
  tt0 = tt_ref[0:1, :]
  tt1 = tt_ref[1:2, :]
  x = w_ref[...] + pos_ref[...] + (tt0 + tf * (tt1 - tt0))
  mean = jnp.mean(x, axis=-1, keepdims=True)
  xc = x - mean
  var = jnp.mean(xc * xc, axis=-1, keepdims=True)
  normed = xc * lax.rsqrt(var + _LN_EPS)
  normed = normed * ls_ref[...] + lb_ref[...]
  o_ref[...] = (
      jnp.dot(normed, W_ref[...], preferred_element_type=jnp.float32)
      + b_ref[...])


def _tc_dense(wrows, pos2d, type_f, type_table, ln_scale, ln_bias, W, b, blk):
  n = wrows.shape[0]
  grid = (n // blk,)
  s_blocks = _S // blk
  return pl.pallas_call(
      _dense_body,
      grid=grid,
      in_specs=[
          pl.BlockSpec((blk, _D), lambda i: (i, 0)),
          pl.BlockSpec((blk, _D), lambda i: (i % s_blocks, 0)),
          pl.BlockSpec((blk, 1), lambda i: (i, 0)),
          pl.BlockSpec((_T, _D), lambda i: (0, 0)),
          pl.BlockSpec((1, _D), lambda i: (0, 0)),
          pl.BlockSpec((1, _D), lambda i: (0, 0)),
          pl.BlockSpec((_D, _M), lambda i: (0, 0)),
          pl.BlockSpec((1, _M), lambda i: (0, 0)),
      ],
      out_specs=pl.BlockSpec((blk, _M), lambda i: (i, 0)),
      out_shape=jax.ShapeDtypeStruct((n, _M), jnp.float32),
      compiler_params=pltpu.CompilerParams(
          dimension_semantics=("arbitrary",)),
  )(wrows, pos2d, type_f, type_table, ln_scale, ln_bias, W, b)


def kernel(input_ids, type_ids, word_table, pos_emb, type_table, ln_scale,
           ln_bias, W, b):
  bs = _B * _S
  ids_flat = input_ids.reshape(bs)
  wrows = _sc_gather(word_table, ids_flat)
  pos2d = pos_emb.reshape(_S, _D)
  type_f = type_ids.reshape(bs, 1).astype(jnp.float32)
  out = _tc_dense(wrows, pos2d, type_f, type_table, ln_scale.reshape(1, _D),
                  ln_bias.reshape(1, _D), W, b.reshape(1, _M), blk=2048)
  return out.reshape(_B, _S, _M)
